# baseline (speedup 1.0000x reference)
import functools

import jax
import jax.numpy as jnp
from jax import lax
from jax.experimental import pallas as pl
from jax.experimental.pallas import tpu as pltpu
from jax.experimental.pallas import tpu_sc as plsc

_RES = 16
_BS = 32
_B = 4096
_NW = 32
_PER_W = _B // _NW
_CH = 16
_NCHUNK = _PER_W // _CH


def _sc_body(myf_hbm, enf_hbm, lkf_hbm, ci_hbm, vf_hbm,
             delta_hbm, cnt_hbm,
             myv, env, lkv, civ, vfv, dbuf, cbuf, idxs, sem0, sem1):
    wid = lax.axis_index("s") * 2 + lax.axis_index("c")
    base = wid * _PER_W

    iota16 = lax.broadcasted_iota(jnp.int32, (16,), 0)
    zero16 = jnp.zeros((16,), jnp.float32)
    sems = (sem0, sem1)

    def _zd(i, carry):
        dbuf[0, pl.ds(i * 16, 16)] = zero16
        dbuf[1, pl.ds(i * 16, 16)] = zero16
        return carry
    lax.fori_loop(0, 1024, _zd, 0)

    def _zc(i, carry):
        cbuf[0, pl.ds(i * 16, 16)] = zero16
        cbuf[1, pl.ds(i * 16, 16)] = zero16
        return carry
    lax.fori_loop(0, 16, _zc, 0)

    def chunk_body(ch, carry):
        cb = base + ch * _CH
        pltpu.sync_copy(myf_hbm.at[pl.ds(cb, _CH)], myv)
        pltpu.sync_copy(enf_hbm.at[pl.ds(cb, _CH)], env)
        pltpu.sync_copy(lkf_hbm.at[pl.ds(cb, _CH)], lkv)
        pltpu.sync_copy(ci_hbm.at[pl.ds(cb, _CH)], civ)
        pltpu.sync_copy(vf_hbm.at[pl.ds(cb, _CH)], vfv)

        def pair_body(pr, carry2):
            for p in range(2):
                sl = pr * 2 + p
                s = cb + sl
                sem = sems[p]

                @pl.when(ch * 8 + pr > 0)
                def _wait_and_rezero():
                    pltpu.make_async_copy(
                        dbuf.at[p], delta_hbm.at[s], sem).wait()
                    pltpu.make_async_copy(
                        cbuf.at[p], cnt_hbm.at[s], sem).wait()

                    def _rz(jj, c3):
                        cc = idxs[p, jj]
                        off = cc * 64
                        for k in range(4):
                            dbuf[p, pl.ds(
                                pl.multiple_of(off + k * 16, 16), 16)] = zero16
                        cbuf[p, pl.ds(
                            pl.multiple_of((cc >> 4) << 4, 16), 16)] = zero16
                        return c3
                    lax.fori_loop(0, 24, _rz, 0)

                ci_a = civ[sl, pl.ds(0, 16)]
                ci_b = civ[sl, pl.ds(16, 16)]
                vf_a = vfv[sl, pl.ds(0, 16)]
                vf_b = vfv[sl, pl.ds(16, 16)]
                for si, ref in enumerate((myv, env, lkv)):
                    for j in range(8):
                        t = si * 8 + j
                        cc = ci_a[t] if t < 16 else ci_b[t - 16]
                        v = vf_a[t] if t < 16 else vf_b[t - 16]
                        off = cc * 64
                        for k in range(4):
                            o16 = pl.multiple_of(off + k * 16, 16)
                            cur = dbuf[p, pl.ds(o16, 16)]
                            dbuf[p, pl.ds(o16, 16)] = (
                                cur + ref[sl, j, pl.ds(k * 16, 16)] * v)
                        lane = jnp.where(iota16 == (cc & 15), v, 0.0)
                        row = pl.multiple_of((cc >> 4) << 4, 16)
                        cbuf[p, pl.ds(row, 16)] = (
                            cbuf[p, pl.ds(row, 16)] + lane)
                        idxs[p, t] = cc

                pltpu.async_copy(dbuf.at[p], delta_hbm.at[s], sem)
                pltpu.async_copy(cbuf.at[p], cnt_hbm.at[s], sem)
            return carry2

        lax.fori_loop(0, _CH // 2, pair_body, 0)
        return carry
    lax.fori_loop(0, _NCHUNK, chunk_body, 0)

    for p in range(2):
        pltpu.make_async_copy(dbuf.at[p], delta_hbm.at[0], sems[p]).wait()
        pltpu.make_async_copy(cbuf.at[p], cnt_hbm.at[0], sems[p]).wait()


def _sc_scatter(myf, enf, lkf, ci, vf):
    mesh = plsc.VectorSubcoreMesh(core_axis_name="c", subcore_axis_name="s")
    f = pl.kernel(
        _sc_body,
        mesh=mesh,
        out_type=[
            jax.ShapeDtypeStruct((_B, 256 * 64), jnp.float32),
            jax.ShapeDtypeStruct((_B, 256), jnp.float32),
        ],
        scratch_types=[
            pltpu.VMEM((_CH, 8, 64), jnp.float32),
            pltpu.VMEM((_CH, 8, 64), jnp.float32),
            pltpu.VMEM((_CH, 8, 64), jnp.float32),
            pltpu.VMEM((_CH, 32), jnp.int32),
            pltpu.VMEM((_CH, 32), jnp.float32),
            pltpu.VMEM((2, 256 * 64), jnp.float32),
            pltpu.VMEM((2, 256), jnp.float32),
            pltpu.SMEM((2, 24), jnp.int32),
            pltpu.SemaphoreType.DMA,
            pltpu.SemaphoreType.DMA,
        ],
    )
    return f(myf, enf, lkf, ci, vf)


def _leaky(x):
    return jnp.where(x >= 0, x, 0.01 * x)


def _conv_s2(x, Wc, b, H, C, K, bs):
    Ho = H // 2
    xr = x.reshape(bs, Ho, 2, H, C)
    even, odd = xr[:, :, 0], xr[:, :, 1]
    rows = {
        0: even,
        1: odd,
        2: jnp.concatenate(
            [even[:, 1:], jnp.zeros_like(even[:, :1])], axis=1),
    }
    taps = []
    for ky in (0, 1, 2):
        yc = rows[ky].reshape(bs, Ho, Ho, 2, C)
        ceven, codd = yc[:, :, :, 0], yc[:, :, :, 1]
        cols = {
            0: ceven,
            1: codd,
            2: jnp.concatenate(
                [ceven[:, :, 1:], jnp.zeros_like(ceven[:, :, :1])], axis=2),
        }
        for kx in (0, 1, 2):
            taps.append(cols[kx])
    big = jnp.concatenate(taps, axis=-1).reshape(bs * Ho * Ho, 9 * C)
    acc = jnp.dot(big, Wc, preferred_element_type=jnp.float32)
    return (acc + b).reshape(bs, Ho, Ho, K)


def _tc_body(g_ref, delta_ref, cnt_ref, myf_ref, lidar_ref,
             w1_ref, b1_ref, w2_ref, b2_ref, w3_ref, b3_ref,
             lns_ref, lnb_ref,
             team_ref, agent_ref):
    bs = _BS
    g = g_ref[...]
    myf = myf_ref[...]

    inv = 1.0 / (1.0 + cnt_ref[...])
    mm = (g[:, None, :] + delta_ref[...]) * inv[:, :, None]
    x = mm.reshape(bs, _RES, _RES, 64)

    o = _leaky(_conv_s2(x, w1_ref[...], b1_ref[...], 16, 64, 32, bs))
    o = _leaky(_conv_s2(o, w2_ref[...], b2_ref[...], 8, 32, 64, bs))
    o = _conv_s2(o, w3_ref[...], b3_ref[...], 4, 64, 64, bs)

    flat = jnp.concatenate(
        [o[:, 0, 0, :], o[:, 0, 1, :], o[:, 1, 0, :], o[:, 1, 1, :]],
        axis=-1)
    mu = jnp.mean(flat, axis=-1, keepdims=True)
    var = jnp.mean(jnp.square(flat - mu), axis=-1, keepdims=True)
    y = (flat - mu) / jnp.sqrt(var + 1e-6) * lns_ref[...] + lnb_ref[...]
    team_ref[...] = _leaky(y)

    agent_ref[...] = jnp.concatenate([myf, lidar_ref[...]], axis=-1)


def _cells(pos):
    ix = jnp.clip((pos[..., 0] * _RES).astype(jnp.int32), 0, _RES - 1)
    iy = jnp.clip((pos[..., 1] * _RES).astype(jnp.int32), 0, _RES - 1)
    return iy * _RES + ix


def kernel(global_features, my_features, my_lidar, enemy_features,
           last_known_enemy_features, my_positions, enemy_positions,
           last_known_enemy_positions, enemy_mask,
           W1, b1, W2, b2, W3, b3, ln_scale, ln_bias, train):
    B, T, D = my_features.shape
    L = my_lidar.shape[-1]
    bs = _BS

    ci = jnp.concatenate(
        [_cells(my_positions), _cells(enemy_positions),
         _cells(last_known_enemy_positions),
         jnp.zeros((B, 8), jnp.int32)], axis=1)
    one = jnp.float32(1.0)
    vf = jnp.concatenate(
        [(my_features[..., 0] == one).astype(jnp.float32),
         jnp.logical_and(enemy_features[..., 0] == one,
                         enemy_mask == one).astype(jnp.float32),
         (last_known_enemy_features[..., 0] == one).astype(jnp.float32),
         jnp.zeros((B, 8), jnp.float32)], axis=1)

    delta, cnt = _sc_scatter(my_features, enemy_features,
                             last_known_enemy_features, ci, vf)
    delta = delta.reshape(B, 256, 64)

    w1r = W1.reshape(9 * D, 32)
    w2r = W2.reshape(9 * 32, 64)
    w3r = W3.reshape(9 * 64, 64)

    def bspec(shape, idx):
        return pl.BlockSpec(shape, idx)

    grid = (B // bs,)
    bmap = lambda i: (i, 0)
    bmap3 = lambda i: (i, 0, 0)
    wmap2 = lambda i: (0, 0)

    in_specs = [
        bspec((bs, D), bmap),
        bspec((bs, 256, 64), bmap3),
        bspec((bs, 256), bmap),
        bspec((bs, T, D), bmap3),
        bspec((bs, T, L), bmap3),
        bspec((9 * D, 32), wmap2),
        bspec((1, 32), wmap2),
        bspec((9 * 32, 64), wmap2),
        bspec((1, 64), wmap2),
        bspec((9 * 64, 64), wmap2),
        bspec((1, 64), wmap2),
        bspec((1, 256), wmap2),
        bspec((1, 256), wmap2),
    ]
    out_specs = [
        bspec((bs, 256), bmap),
        bspec((bs, T, D + L), bmap3),
    ]
    out_shape = [
        jax.ShapeDtypeStruct((B, 256), jnp.float32),
        jax.ShapeDtypeStruct((B, T, D + L), jnp.float32),
    ]

    team, agent = pl.pallas_call(
        _tc_body,
        grid=grid,
        in_specs=in_specs,
        out_specs=out_specs,
        out_shape=out_shape,
    )(
        global_features, delta, cnt, my_features, my_lidar,
        w1r, b1.reshape(1, 32), w2r, b2.reshape(1, 64),
        w3r, b3.reshape(1, 64),
        ln_scale.reshape(1, 256), ln_bias.reshape(1, 256),
    )
    return (team, agent)


# speedup vs baseline: 1.0873x; 1.0873x over previous
import functools

import jax
import jax.numpy as jnp
from jax import lax
from jax.experimental import pallas as pl
from jax.experimental.pallas import tpu as pltpu
from jax.experimental.pallas import tpu_sc as plsc

_RES = 16
_BS = 32
_B = 4096
_NW = 32
_PER_W = _B // _NW
_CH = 16
_NCHUNK = _PER_W // _CH


def _sc_body(gf_hbm, myf_hbm, enf_hbm, lkf_hbm, ci_hbm, vf_hbm,
             corr_hbm,
             gv, myv, env, lkv, civ, vfv, dbuf, nbuf, idxs, sem0, sem1):
    wid = lax.axis_index("s") * 2 + lax.axis_index("c")
    base = wid * _PER_W

    iota16 = lax.broadcasted_iota(jnp.int32, (16,), 0)
    zero16 = jnp.zeros((16,), jnp.float32)
    sems = (sem0, sem1)

    def _zd(i, carry):
        dbuf[0, pl.ds(i * 16, 16)] = zero16
        dbuf[1, pl.ds(i * 16, 16)] = zero16
        return carry
    lax.fori_loop(0, 1024, _zd, 0)

    def _zc(i, carry):
        nbuf[0, pl.ds(i * 16, 16)] = zero16
        nbuf[1, pl.ds(i * 16, 16)] = zero16
        return carry
    lax.fori_loop(0, 1024, _zc, 0)

    def chunk_body(ch, carry):
        cb = base + ch * _CH
        pltpu.sync_copy(gf_hbm.at[pl.ds(cb, _CH)], gv)
        pltpu.sync_copy(myf_hbm.at[pl.ds(cb, _CH)], myv)
        pltpu.sync_copy(enf_hbm.at[pl.ds(cb, _CH)], env)
        pltpu.sync_copy(lkf_hbm.at[pl.ds(cb, _CH)], lkv)
        pltpu.sync_copy(ci_hbm.at[pl.ds(cb, _CH)], civ)
        pltpu.sync_copy(vf_hbm.at[pl.ds(cb, _CH)], vfv)

        def pair_body(pr, carry2):
            for p in range(2):
                sl = pr * 2 + p
                s = cb + sl
                sem = sems[p]

                @pl.when(ch * (_CH // 2) + pr > 0)
                def _wait_and_rezero():
                    pltpu.make_async_copy(
                        dbuf.at[p], corr_hbm.at[s], sem).wait()

                    def _rz(jj, c3):
                        cc = idxs[p, jj]
                        off = (cc & 127) * 128 + (cc >> 7) * 64
                        for k in range(4):
                            dbuf[p, pl.ds(
                                pl.multiple_of(off + k * 16, 16), 16)] = zero16
                        return c3
                    lax.fori_loop(0, 24, _rz, 0)

                ci_a = civ[sl, pl.ds(0, 16)]
                ci_b = civ[sl, pl.ds(16, 16)]
                vf_a = vfv[sl, pl.ds(0, 16)]
                vf_b = vfv[sl, pl.ds(16, 16)]
                for si, ref in enumerate((myv, env, lkv)):
                    for j in range(8):
                        t = si * 8 + j
                        cc = ci_a[t] if t < 16 else ci_b[t - 16]
                        v = vf_a[t] if t < 16 else vf_b[t - 16]
                        off = (cc & 127) * 128 + (cc >> 7) * 64
                        for k in range(4):
                            o16 = pl.multiple_of(off + k * 16, 16)
                            cur = dbuf[p, pl.ds(o16, 16)]
                            dbuf[p, pl.ds(o16, 16)] = (
                                cur + ref[sl, j, pl.ds(k * 16, 16)] * v)
                            curn = nbuf[p, pl.ds(o16, 16)]
                            nbuf[p, pl.ds(o16, 16)] = curn + v
                        idxs[p, t] = cc

                for t in range(24):
                    cc = idxs[p, t]
                    off = (cc & 127) * 128 + (cc >> 7) * 64
                    for k in range(4):
                        o16 = pl.multiple_of(off + k * 16, 16)
                        nk = nbuf[p, pl.ds(o16, 16)]
                        d = dbuf[p, pl.ds(o16, 16)]
                        gk = gv[sl, pl.ds(k * 16, 16)]
                        dbuf[p, pl.ds(o16, 16)] = (
                            (d - nk * gk) / (1.0 + nk))
                        nbuf[p, pl.ds(o16, 16)] = zero16

                pltpu.async_copy(dbuf.at[p], corr_hbm.at[s], sem)
            return carry2

        lax.fori_loop(0, _CH // 2, pair_body, 0)
        return carry
    lax.fori_loop(0, _NCHUNK, chunk_body, 0)

    for p in range(2):
        pltpu.make_async_copy(dbuf.at[p], corr_hbm.at[0], sems[p]).wait()


def _sc_scatter(gf, myf, enf, lkf, ci, vf):
    mesh = plsc.VectorSubcoreMesh(core_axis_name="c", subcore_axis_name="s")
    f = pl.kernel(
        _sc_body,
        mesh=mesh,
        out_type=[
            jax.ShapeDtypeStruct((_B, 128 * 128), jnp.float32),
        ],
        scratch_types=[
            pltpu.VMEM((_CH, 64), jnp.float32),
            pltpu.VMEM((_CH, 8, 64), jnp.float32),
            pltpu.VMEM((_CH, 8, 64), jnp.float32),
            pltpu.VMEM((_CH, 8, 64), jnp.float32),
            pltpu.VMEM((_CH, 32), jnp.int32),
            pltpu.VMEM((_CH, 32), jnp.float32),
            pltpu.VMEM((2, 128 * 128), jnp.float32),
            pltpu.VMEM((2, 128 * 128), jnp.float32),
            pltpu.SMEM((2, 24), jnp.int32),
            pltpu.SemaphoreType.DMA,
            pltpu.SemaphoreType.DMA,
        ],
    )
    return f(gf, myf, enf, lkf, ci, vf)


def _leaky(x):
    return jnp.where(x >= 0, x, 0.01 * x)


def _conv_s2(x, Wc, b, H, C, K, bs):
    Ho = H // 2
    xr = x.reshape(bs, Ho, 2, H, C)
    even, odd = xr[:, :, 0], xr[:, :, 1]
    rows = {
        0: even,
        1: odd,
        2: jnp.concatenate(
            [even[:, 1:], jnp.zeros_like(even[:, :1])], axis=1),
    }
    taps = []
    for ky in (0, 1, 2):
        yc = rows[ky].reshape(bs, Ho, Ho, 2, C)
        ceven, codd = yc[:, :, :, 0], yc[:, :, :, 1]
        cols = {
            0: ceven,
            1: codd,
            2: jnp.concatenate(
                [ceven[:, :, 1:], jnp.zeros_like(ceven[:, :, :1])], axis=2),
        }
        for kx in (0, 1, 2):
            taps.append(cols[kx])
    big = jnp.concatenate(taps, axis=-1).reshape(bs * Ho * Ho, 9 * C)
    acc = jnp.dot(big, Wc, preferred_element_type=jnp.float32)
    return (acc + b).reshape(bs, Ho, Ho, K)


def _tc_body(g_ref, corr_ref, myf_ref, lidar_ref,
             w1_ref, b1_ref, w2_ref, b2_ref, w3_ref, b3_ref,
             lns_ref, lnb_ref,
             team_ref, agent_ref):
    bs = _BS
    g = g_ref[...]
    myf = myf_ref[...]

    corr = corr_ref[...]
    mm_top = corr[:, :, 0:64] + g[:, None, :]
    mm_bot = corr[:, :, 64:128] + g[:, None, :]
    x = jnp.concatenate(
        [mm_top.reshape(bs, 8, _RES, 64), mm_bot.reshape(bs, 8, _RES, 64)],
        axis=1)

    o = _leaky(_conv_s2(x, w1_ref[...], b1_ref[...], 16, 64, 32, bs))
    o = _leaky(_conv_s2(o, w2_ref[...], b2_ref[...], 8, 32, 64, bs))
    o = _conv_s2(o, w3_ref[...], b3_ref[...], 4, 64, 64, bs)

    flat = jnp.concatenate(
        [o[:, 0, 0, :], o[:, 0, 1, :], o[:, 1, 0, :], o[:, 1, 1, :]],
        axis=-1)
    mu = jnp.mean(flat, axis=-1, keepdims=True)
    var = jnp.mean(jnp.square(flat - mu), axis=-1, keepdims=True)
    y = (flat - mu) / jnp.sqrt(var + 1e-6) * lns_ref[...] + lnb_ref[...]
    team_ref[...] = _leaky(y)

    agent_ref[...] = jnp.concatenate([myf, lidar_ref[...]], axis=-1)


def _cells(pos):
    ix = jnp.clip((pos[..., 0] * _RES).astype(jnp.int32), 0, _RES - 1)
    iy = jnp.clip((pos[..., 1] * _RES).astype(jnp.int32), 0, _RES - 1)
    return iy * _RES + ix


def kernel(global_features, my_features, my_lidar, enemy_features,
           last_known_enemy_features, my_positions, enemy_positions,
           last_known_enemy_positions, enemy_mask,
           W1, b1, W2, b2, W3, b3, ln_scale, ln_bias, train):
    B, T, D = my_features.shape
    L = my_lidar.shape[-1]
    bs = _BS

    ci = jnp.concatenate(
        [_cells(my_positions), _cells(enemy_positions),
         _cells(last_known_enemy_positions),
         jnp.zeros((B, 8), jnp.int32)], axis=1)
    one = jnp.float32(1.0)
    vf = jnp.concatenate(
        [(my_features[..., 0] == one).astype(jnp.float32),
         jnp.logical_and(enemy_features[..., 0] == one,
                         enemy_mask == one).astype(jnp.float32),
         (last_known_enemy_features[..., 0] == one).astype(jnp.float32),
         jnp.zeros((B, 8), jnp.float32)], axis=1)

    corr, = _sc_scatter(global_features, my_features, enemy_features,
                        last_known_enemy_features, ci, vf)
    corr = corr.reshape(B, 128, 128)

    w1r = W1.reshape(9 * D, 32)
    w2r = W2.reshape(9 * 32, 64)
    w3r = W3.reshape(9 * 64, 64)

    def bspec(shape, idx):
        return pl.BlockSpec(shape, idx)

    grid = (B // bs,)
    bmap = lambda i: (i, 0)
    bmap3 = lambda i: (i, 0, 0)
    wmap2 = lambda i: (0, 0)

    in_specs = [
        bspec((bs, D), bmap),
        bspec((bs, 128, 128), bmap3),
        bspec((bs, T, D), bmap3),
        bspec((bs, T, L), bmap3),
        bspec((9 * D, 32), wmap2),
        bspec((1, 32), wmap2),
        bspec((9 * 32, 64), wmap2),
        bspec((1, 64), wmap2),
        bspec((9 * 64, 64), wmap2),
        bspec((1, 64), wmap2),
        bspec((1, 256), wmap2),
        bspec((1, 256), wmap2),
    ]
    out_specs = [
        bspec((bs, 256), bmap),
        bspec((bs, T, D + L), bmap3),
    ]
    out_shape = [
        jax.ShapeDtypeStruct((B, 256), jnp.float32),
        jax.ShapeDtypeStruct((B, T, D + L), jnp.float32),
    ]

    team, agent = pl.pallas_call(
        _tc_body,
        grid=grid,
        in_specs=in_specs,
        out_specs=out_specs,
        out_shape=out_shape,
    )(
        global_features, corr, my_features, my_lidar,
        w1r, b1.reshape(1, 32), w2r, b2.reshape(1, 64),
        w3r, b3.reshape(1, 64),
        ln_scale.reshape(1, 256), ln_bias.reshape(1, 256),
    )
    return (team, agent)
